# baseline (speedup 1.0000x reference)
import functools

import jax
import jax.numpy as jnp
from jax import lax
from jax.experimental import pallas as pl
from jax.experimental.pallas import tpu as pltpu
from jax.experimental.pallas import tpu_sc as plsc


def _project_body(cbfv_ref, w_ref, b_ref, out_ref):
    acc = lax.dot_general(
        cbfv_ref[...], w_ref[...],
        dimension_numbers=(((1,), (1,)), ((), ())),
        preferred_element_type=jnp.float32,
    )
    out_ref[...] = acc + b_ref[...][None, :]


def _project_table(cbfv, W, b):
    vocab = cbfv.shape[0]
    d_model = W.shape[0]
    return pl.pallas_call(
        _project_body,
        out_shape=jax.ShapeDtypeStruct((vocab, d_model), jnp.float32),
    )(cbfv, W, b)


_CB = 1
_NBUF = 2
_LANES = 16


@functools.partial(jax.jit, static_argnums=(2, 3, 4))
def _sc_gather(table_flat, idx, batch, seq, d_model):
    try:
        info = plsc.get_sparse_core_info()
        nc, ns = info.num_cores, info.num_subcores
    except Exception:
        nc, ns = 2, 16
    nw = nc * ns
    assert batch % (nw * _CB) == 0 and d_model % _LANES == 0
    b_per_w = batch // nw
    rows_per_w = b_per_w * seq
    rows_per_chunk = _CB * seq
    n_chunks = b_per_w // _CB
    assert n_chunks % _NBUF == 0 and n_chunks >= 2 * _NBUF
    n_col = d_model // _LANES

    mesh = plsc.VectorSubcoreMesh(core_axis_name="c", subcore_axis_name="s")

    @functools.partial(
        pl.kernel,
        mesh=mesh,
        out_type=jax.ShapeDtypeStruct((batch, seq, d_model), jnp.float32),
        scratch_types=[
            pltpu.VMEM((table_flat.shape[0],), jnp.float32),
            pltpu.VMEM((rows_per_w,), jnp.int32),
        ] + [pltpu.VMEM((_CB, seq, d_model), jnp.float32)] * _NBUF
          + [pltpu.SemaphoreType.DMA] * _NBUF,
        compiler_params=pltpu.CompilerParams(needs_layout_passes=False),
    )
    def gather_kernel(table_hbm, idx_hbm, out_hbm, table_v, idx_v, *rest):
        bufs = rest[:_NBUF]
        osems = rest[_NBUF:]
        wid = lax.axis_index("s") * nc + lax.axis_index("c")
        rbase = pl.multiple_of(wid * rows_per_w, rows_per_w)
        bbase = pl.multiple_of(wid * b_per_w, b_per_w)
        pltpu.sync_copy(table_hbm, table_v)
        pltpu.sync_copy(idx_hbm.at[pl.ds(rbase, rows_per_w)], idx_v)

        lanes = lax.iota(jnp.int32, _LANES)

        def fill(j, b):
            roff = j * rows_per_chunk
            for r in range(rows_per_chunk):
                pos = jnp.full((_LANES,), roff + r, dtype=jnp.int32)
                rowvec = plsc.load_gather(idx_v, [pos])
                addr0 = rowvec * d_model + lanes
                kb, s = divmod(r, seq)
                for c in range(n_col):
                    v = plsc.load_gather(table_v, [addr0 + (c * _LANES)])
                    bufs[b][kb, s, pl.ds(c * _LANES, _LANES)] = v

        def start_out(j, b):
            pltpu.async_copy(bufs[b],
                             out_hbm.at[pl.ds(bbase + j * _CB, _CB)], osems[b])

        def wait_out(b):
            pltpu.make_async_copy(
                bufs[b], out_hbm.at[pl.ds(bbase, _CB)], osems[b]).wait()

        fill(0, 0)
        start_out(0, 0)
        fill(1, 1)
        start_out(1, 1)

        def body(g, carry):
            for d in range(_NBUF):
                j = _NBUF * (g + 1) + d
                b = d
                wait_out(b)
                fill(j, b)
                start_out(j, b)
            return carry

        lax.fori_loop(0, n_chunks // _NBUF - 1, body, 0)

        for b in range(_NBUF):
            wait_out(b)

    return gather_kernel(table_flat, idx)


def kernel(src, cbfv, W, b):
    batch, seq = src.shape
    d_model = W.shape[0]
    table = _project_table(cbfv, W, b)
    idx = src.reshape(-1).astype(jnp.int32)
    return _sc_gather(table.reshape(-1), idx, batch, seq, d_model)


# speedup vs baseline: 1.3135x; 1.3135x over previous
import functools

import jax
import jax.numpy as jnp
from jax import lax
from jax.experimental import pallas as pl
from jax.experimental.pallas import tpu as pltpu
from jax.experimental.pallas import tpu_sc as plsc


def _project_body(cbfv_ref, w_ref, b_ref, out_ref):
    acc = lax.dot_general(
        cbfv_ref[...], w_ref[...],
        dimension_numbers=(((1,), (1,)), ((), ())),
        preferred_element_type=jnp.float32,
    )
    out_ref[...] = acc + b_ref[...][None, :]


def _project_table(cbfv, W, b):
    vocab = cbfv.shape[0]
    d_model = W.shape[0]
    return pl.pallas_call(
        _project_body,
        out_shape=jax.ShapeDtypeStruct((vocab, d_model), jnp.float32),
    )(cbfv, W, b)


_CB = 1
_NBUF = 2
_LANES = 16


@functools.partial(jax.jit, static_argnums=(2, 3, 4))
def _sc_gather(table_flat, idx, batch, seq, d_model):
    try:
        info = plsc.get_sparse_core_info()
        nc, ns = info.num_cores, info.num_subcores
    except Exception:
        nc, ns = 2, 16
    nw = nc * ns
    assert batch % (nw * _CB) == 0 and d_model % _LANES == 0
    b_per_w = batch // nw
    rows_per_w = b_per_w * seq
    rows_per_chunk = _CB * seq
    n_chunks = b_per_w // _CB
    assert n_chunks % _NBUF == 0 and n_chunks >= 2 * _NBUF
    n_col = d_model // _LANES

    mesh = plsc.VectorSubcoreMesh(core_axis_name="c", subcore_axis_name="s",
                                  num_cores=nc, num_subcores=ns)

    @functools.partial(
        pl.kernel,
        mesh=mesh,
        out_type=jax.ShapeDtypeStruct((batch, seq, d_model), jnp.float32),
        scratch_types=[
            pltpu.VMEM((table_flat.shape[0],), jnp.float32),
            pltpu.VMEM((rows_per_w,), jnp.int32),
        ] + [pltpu.VMEM((_CB, seq, d_model), jnp.float32)] * _NBUF
          + [pltpu.SemaphoreType.DMA] * _NBUF,
        compiler_params=pltpu.CompilerParams(needs_layout_passes=False),
    )
    def gather_kernel(table_hbm, idx_hbm, out_hbm, table_v, idx_v, *rest):
        bufs = rest[:_NBUF]
        osems = rest[_NBUF:]
        wid = lax.axis_index("s") * nc + lax.axis_index("c")
        rbase = pl.multiple_of(wid * rows_per_w, rows_per_w)
        bbase = pl.multiple_of(wid * b_per_w, b_per_w)
        pltpu.sync_copy(table_hbm, table_v)
        pltpu.sync_copy(idx_hbm.at[pl.ds(rbase, rows_per_w)], idx_v)

        lanes = lax.iota(jnp.int32, _LANES)

        def fill(j, b):
            roff = j * rows_per_chunk
            for r in range(rows_per_chunk):
                pos = jnp.full((_LANES,), roff + r, dtype=jnp.int32)
                rowvec = plsc.load_gather(idx_v, [pos])
                addr0 = rowvec * d_model + lanes
                kb, s = divmod(r, seq)
                for c in range(n_col):
                    v = plsc.load_gather(table_v, [addr0 + (c * _LANES)])
                    bufs[b][kb, s, pl.ds(c * _LANES, _LANES)] = v

        def start_out(j, b):
            pltpu.async_copy(bufs[b],
                             out_hbm.at[pl.ds(bbase + j * _CB, _CB)], osems[b])

        def wait_out(b):
            pltpu.make_async_copy(
                bufs[b], out_hbm.at[pl.ds(bbase, _CB)], osems[b]).wait()
            plsc.subcore_barrier()

        def body(g, carry):
            for d in range(_NBUF):
                j = g * _NBUF + d
                b = d

                @pl.when(g > 0)
                def _():
                    wait_out(b)

                fill(j, b)
                start_out(j, b)
            return carry

        lax.fori_loop(0, n_chunks // _NBUF, body, 0)

        for b in range(_NBUF):
            wait_out(b)

    return gather_kernel(table_flat, idx)


def kernel(src, cbfv, W, b):
    batch, seq = src.shape
    d_model = W.shape[0]
    table = _project_table(cbfv, W, b)
    idx = src.reshape(-1).astype(jnp.int32)
    return _sc_gather(table.reshape(-1), idx, batch, seq, d_model)


# speedup vs baseline: 2.6683x; 2.0314x over previous
import functools

import jax
import jax.numpy as jnp
from jax import lax
from jax.experimental import pallas as pl
from jax.experimental.pallas import tpu as pltpu
from jax.experimental.pallas import tpu_sc as plsc


def _project_body(cbfv_ref, w_ref, b_ref, out_ref):
    acc = lax.dot_general(
        cbfv_ref[...], w_ref[...],
        dimension_numbers=(((1,), (1,)), ((), ())),
        preferred_element_type=jnp.float32,
    )
    out_ref[...] = acc + b_ref[...][None, :]


def _project_table(cbfv, W, b):
    vocab = cbfv.shape[0]
    d_model = W.shape[0]
    return pl.pallas_call(
        _project_body,
        out_shape=jax.ShapeDtypeStruct((vocab, d_model), jnp.float32),
    )(cbfv, W, b)


_CB = 2
_NBUF = 2
_LANES = 16


@functools.partial(jax.jit, static_argnums=(2, 3, 4))
def _sc_gather(table_flat, idx, batch, seq, d_model):
    try:
        info = plsc.get_sparse_core_info()
        nc, ns = info.num_cores, info.num_subcores
    except Exception:
        nc, ns = 2, 16
    nw = nc * ns
    assert batch % (nw * _CB) == 0 and d_model % _LANES == 0
    b_per_w = batch // nw
    rows_per_w = b_per_w * seq
    rows_per_chunk = _CB * seq
    n_chunks = b_per_w // _CB
    assert n_chunks % _NBUF == 0 and n_chunks >= 2 * _NBUF
    n_col = d_model // _LANES

    mesh = plsc.VectorSubcoreMesh(core_axis_name="c", subcore_axis_name="s",
                                  num_cores=nc, num_subcores=ns)

    @functools.partial(
        pl.kernel,
        mesh=mesh,
        out_type=jax.ShapeDtypeStruct((batch, seq, d_model), jnp.float32),
        scratch_types=[
            pltpu.VMEM((table_flat.shape[0],), jnp.float32),
            pltpu.VMEM((rows_per_w,), jnp.int32),
        ] + [pltpu.VMEM((_CB, seq, d_model), jnp.float32)] * _NBUF
          + [pltpu.SemaphoreType.DMA] * _NBUF,
        compiler_params=pltpu.CompilerParams(needs_layout_passes=False),
    )
    def gather_kernel(table_hbm, idx_hbm, out_hbm, table_v, idx_v, *rest):
        bufs = rest[:_NBUF]
        osems = rest[_NBUF:]
        wid = lax.axis_index("s") * nc + lax.axis_index("c")
        rbase = pl.multiple_of(wid * rows_per_w, rows_per_w)
        bbase = pl.multiple_of(wid * b_per_w, b_per_w)
        pltpu.sync_copy(table_hbm, table_v)
        pltpu.sync_copy(idx_hbm.at[pl.ds(rbase, rows_per_w)], idx_v)

        lanes = lax.iota(jnp.int32, _LANES)

        def fill(j, b):
            roff = j * rows_per_chunk
            buf = bufs[b]

            @plsc.parallel_loop(0, rows_per_chunk, unroll=2)
            def _(r):
                pos = jnp.full((_LANES,), roff + r, dtype=jnp.int32)
                rowvec = plsc.load_gather(idx_v, [pos])
                addr0 = rowvec * d_model + lanes
                kb_vec = jnp.full((_LANES,), r // seq, dtype=jnp.int32)
                s_vec = jnp.full((_LANES,), r % seq, dtype=jnp.int32)
                for c in range(n_col):
                    v = plsc.load_gather(table_v, [addr0 + (c * _LANES)])
                    plsc.store_scatter(buf, [kb_vec, s_vec, lanes + (c * _LANES)], v)

        def start_out(j, b):
            pltpu.async_copy(bufs[b],
                             out_hbm.at[pl.ds(bbase + j * _CB, _CB)], osems[b])

        def wait_out(b):
            pltpu.make_async_copy(
                bufs[b], out_hbm.at[pl.ds(bbase, _CB)], osems[b]).wait()
            plsc.subcore_barrier()

        def body(g, carry):
            for d in range(_NBUF):
                j = g * _NBUF + d
                b = d

                @pl.when(g > 0)
                def _():
                    wait_out(b)

                fill(j, b)
                start_out(j, b)
            return carry

        lax.fori_loop(0, n_chunks // _NBUF, body, 0)

        for b in range(_NBUF):
            wait_out(b)

    return gather_kernel(table_flat, idx)


def kernel(src, cbfv, W, b):
    batch, seq = src.shape
    d_model = W.shape[0]
    table = _project_table(cbfv, W, b)
    idx = src.reshape(-1).astype(jnp.int32)
    return _sc_gather(table.reshape(-1), idx, batch, seq, d_model)


# speedup vs baseline: 2.7007x; 1.0122x over previous
import functools

import jax
import jax.numpy as jnp
from jax import lax
from jax.experimental import pallas as pl
from jax.experimental.pallas import tpu as pltpu
from jax.experimental.pallas import tpu_sc as plsc


def _project_body(cbfv_ref, w_ref, b_ref, out_ref):
    acc = lax.dot_general(
        cbfv_ref[...], w_ref[...],
        dimension_numbers=(((1,), (1,)), ((), ())),
        preferred_element_type=jnp.float32,
    )
    out_ref[...] = acc + b_ref[...][None, :]


def _project_table(cbfv, W, b):
    vocab = cbfv.shape[0]
    d_model = W.shape[0]
    return pl.pallas_call(
        _project_body,
        out_shape=jax.ShapeDtypeStruct((vocab, d_model), jnp.float32),
    )(cbfv, W, b)


_CB = 2
_NBUF = 2
_LANES = 16


@functools.partial(jax.jit, static_argnums=(2, 3, 4))
def _sc_gather(table_flat, idx, batch, seq, d_model):
    try:
        info = plsc.get_sparse_core_info()
        nc, ns = info.num_cores, info.num_subcores
    except Exception:
        nc, ns = 2, 16
    nw = nc * ns
    assert batch % (nw * _CB) == 0 and d_model % _LANES == 0
    b_per_w = batch // nw
    rows_per_w = b_per_w * seq
    rows_per_chunk = _CB * seq
    n_chunks = b_per_w // _CB
    assert n_chunks % _NBUF == 0 and n_chunks >= 2 * _NBUF
    n_col = d_model // _LANES

    mesh = plsc.VectorSubcoreMesh(core_axis_name="c", subcore_axis_name="s",
                                  num_cores=nc, num_subcores=ns)

    @functools.partial(
        pl.kernel,
        mesh=mesh,
        out_type=jax.ShapeDtypeStruct((batch, seq, d_model), jnp.float32),
        scratch_types=[
            pltpu.VMEM((table_flat.shape[0],), jnp.float32),
            pltpu.VMEM((rows_per_w,), jnp.int32),
        ] + [pltpu.VMEM((_CB, seq, d_model), jnp.float32)] * _NBUF
          + [pltpu.SemaphoreType.DMA] * _NBUF,
        compiler_params=pltpu.CompilerParams(needs_layout_passes=False),
    )
    def gather_kernel(table_hbm, idx_hbm, out_hbm, table_v, idx_v, *rest):
        bufs = rest[:_NBUF]
        osems = rest[_NBUF:]
        wid = lax.axis_index("s") * nc + lax.axis_index("c")
        rbase = pl.multiple_of(wid * rows_per_w, rows_per_w)
        bbase = pl.multiple_of(wid * b_per_w, b_per_w)
        pltpu.sync_copy(table_hbm, table_v)
        pltpu.sync_copy(idx_hbm.at[pl.ds(rbase, rows_per_w)], idx_v)

        lanes = lax.iota(jnp.int32, _LANES)

        def fill(j, b):
            roff = j * rows_per_chunk
            buf = bufs[b]

            @plsc.parallel_loop(0, rows_per_chunk, unroll=2)
            def _(r):
                pos = jnp.full((_LANES,), roff + r, dtype=jnp.int32)
                rowvec = plsc.load_gather(idx_v, [pos])
                addr0 = rowvec * d_model + lanes
                kb_vec = jnp.full((_LANES,), r // seq, dtype=jnp.int32)
                s_vec = jnp.full((_LANES,), r % seq, dtype=jnp.int32)
                for c in range(n_col):
                    v = plsc.load_gather(table_v, [addr0 + (c * _LANES)])
                    plsc.store_scatter(buf, [kb_vec, s_vec, lanes + (c * _LANES)], v)

        def start_out(j, b):
            pltpu.async_copy(bufs[b],
                             out_hbm.at[pl.ds(bbase + j * _CB, _CB)], osems[b])

        def wait_out(b):
            pltpu.make_async_copy(
                bufs[b], out_hbm.at[pl.ds(bbase, _CB)], osems[b]).wait()

        def body(g, carry):
            for d in range(_NBUF):
                j = g * _NBUF + d
                b = d

                @pl.when(g > 0)
                def _():
                    wait_out(b)

                fill(j, b)
                start_out(j, b)
            return carry

        lax.fori_loop(0, n_chunks // _NBUF, body, 0)

        for b in range(_NBUF):
            wait_out(b)

    return gather_kernel(table_flat, idx)


def kernel(src, cbfv, W, b):
    batch, seq = src.shape
    d_model = W.shape[0]
    table = _project_table(cbfv, W, b)
    idx = src.reshape(-1).astype(jnp.int32)
    return _sc_gather(table.reshape(-1), idx, batch, seq, d_model)
